# baseline (speedup 1.0000x reference)
import jax
import jax.numpy as jnp
from jax import lax
from jax.experimental import pallas as pl
from jax.experimental.pallas import tpu as pltpu
from jax.experimental.pallas import tpu_sc as plsc

_EMBED = 128
_CHUNK = 128


def kernel(timestep, time_embs):
    batch = timestep.shape[0]
    mesh = plsc.VectorSubcoreMesh(core_axis_name="c", subcore_axis_name="s")
    nw = mesh.num_cores * mesh.num_subcores
    b_per_w = batch // nw
    n_chunks = b_per_w // _CHUNK
    idx2d = timestep.reshape((nw * n_chunks, _CHUNK))

    @pl.kernel(
        out_type=jax.ShapeDtypeStruct((batch, _EMBED), time_embs.dtype),
        mesh=mesh,
        scratch_types=[
            pltpu.VMEM((n_chunks, _CHUNK), jnp.int32),
            pltpu.VMEM((n_chunks, _CHUNK, _EMBED), jnp.float32),
            pltpu.SemaphoreType.DMA((n_chunks,)),
            pltpu.SemaphoreType.DMA((n_chunks,)),
        ],
    )
    def _gather(table_hbm, idx_hbm, out_hbm, idx_v, buf_v, gsem, wsem):
        wid = lax.axis_index("s") * mesh.num_cores + lax.axis_index("c")
        pltpu.sync_copy(idx_hbm.at[pl.ds(wid * n_chunks, n_chunks)], idx_v)

        gathers = []
        for j in range(n_chunks):
            gathers.append(pltpu.async_copy(
                table_hbm.at[idx_v.at[j]], buf_v.at[j], gsem.at[j]))
        writes = []
        for j in range(n_chunks):
            gathers[j].wait()
            dst = out_hbm.at[pl.ds((wid * n_chunks + j) * _CHUNK, _CHUNK)]
            writes.append(pltpu.async_copy(buf_v.at[j], dst, wsem.at[j]))
        for w in writes:
            w.wait()

    return _gather(time_embs, idx2d)
